# baseline (speedup 1.0000x reference)
import functools

import jax
import jax.numpy as jnp
from jax import lax
from jax.experimental import pallas as pl
from jax.experimental.pallas import tpu as pltpu
from jax.experimental.pallas import tpu_sc as plsc

NC = 2
NS = 16
NW = NC * NS
D = 128
BW = 128
CH = 64
SUB = BW // CH
LEAD = 5
NBUF = 2 * LEAD


def _build(b, s):
    assert b == NW * BW
    chunks = s * SUB
    assert chunks % NBUF == 0

    mesh = plsc.VectorSubcoreMesh(core_axis_name="c", subcore_axis_name="s")

    @functools.partial(
        pl.kernel,
        out_type=jax.ShapeDtypeStruct((s, b, D), jnp.float32),
        mesh=mesh,
        scratch_types=[
            pltpu.VMEM((s, BW), jnp.int32),
            pltpu.VMEM((NBUF, CH, D), jnp.float32),
        ] + [pltpu.SemaphoreType.DMA] * (2 * NBUF),
    )
    def gather_kernel(idx_hbm, table_hbm, out_hbm, idx_v, rows_v, *sems):
        gsem = sems[:NBUF]
        wsem = sems[NBUF:]
        wid = lax.axis_index("s") * NC + lax.axis_index("c")
        col0 = wid * BW
        pltpu.sync_copy(idx_hbm.at[:, pl.ds(col0, BW)], idx_v)

        def gather(t, buf):
            j = t // SUB
            h = t % SUB
            return pltpu.make_async_copy(
                table_hbm.at[idx_v.at[j, pl.ds(h * CH, CH)]],
                rows_v.at[buf],
                gsem[buf],
            )

        def write(t, buf):
            j = t // SUB
            h = t % SUB
            return pltpu.make_async_copy(
                rows_v.at[buf],
                out_hbm.at[j, pl.ds(col0 + h * CH, CH)],
                wsem[buf],
            )

        for t in range(LEAD):
            gather(t, t).start()

        def group(i, carry):
            t0 = i * NBUF
            for k in range(NBUF):
                t = t0 + k
                gather(t, k).wait()
                write(t, k).start()
                bb = (k + LEAD) % NBUF

                @pl.when(t >= LEAD)
                def _():
                    write(t - LEAD, bb).wait()

                @pl.when(t + LEAD < chunks)
                def _():
                    gather(t + LEAD, bb).start()
            return carry

        lax.fori_loop(0, chunks // NBUF, group, None)

        for t in range(chunks - LEAD, chunks):
            write(t, t % NBUF).wait()

    return gather_kernel


@jax.jit
def kernel(x, table):
    b, s = x.shape
    p = _build(b, s)(x.T, table)
    return jnp.transpose(p, (1, 0, 2))


# speedup vs baseline: 1.0006x; 1.0006x over previous
import functools

import jax
import jax.numpy as jnp
from jax import lax
from jax.experimental import pallas as pl
from jax.experimental.pallas import tpu as pltpu
from jax.experimental.pallas import tpu_sc as plsc

NC = 2
NS = 16
NW = NC * NS
D = 128
BW = 128
NBUF = 5


def _build(b, s):
    assert b == NW * BW and s % NBUF == 0

    mesh = plsc.VectorSubcoreMesh(core_axis_name="c", subcore_axis_name="s")

    @functools.partial(
        pl.kernel,
        out_type=jax.ShapeDtypeStruct((s, b, D), jnp.float32),
        mesh=mesh,
        scratch_types=[
            pltpu.VMEM((s, BW), jnp.int32),
            pltpu.VMEM((NBUF, BW, D), jnp.float32),
        ] + [pltpu.SemaphoreType.DMA] * NBUF,
    )
    def gather_kernel(idx_hbm, table_hbm, out_hbm, idx_v, rows_v, *sems):
        wid = lax.axis_index("s") * NC + lax.axis_index("c")
        col0 = wid * BW
        pltpu.sync_copy(idx_hbm.at[:, pl.ds(col0, BW)], idx_v)

        def gather(g, buf):
            return pltpu.make_async_copy(
                table_hbm.at[idx_v.at[g]], rows_v.at[buf], sems[buf]
            )

        for buf in range(NBUF):
            gather(buf, buf).start()

        def group(i, carry):
            g0 = i * NBUF
            for buf in range(NBUF):
                g = g0 + buf
                gather(g, buf).wait()
                pltpu.sync_copy(rows_v.at[buf], out_hbm.at[g, pl.ds(col0, BW)])
                nxt = g + NBUF

                @pl.when(nxt < s)
                def _():
                    gather(nxt, buf).start()
            return carry

        lax.fori_loop(0, s // NBUF, group, None)

    return gather_kernel


@jax.jit
def kernel(x, table):
    b, s = x.shape
    p = _build(b, s)(x.T, table)
    return jnp.transpose(p, (1, 0, 2))


# speedup vs baseline: 1.0055x; 1.0049x over previous
import functools

import jax
import jax.numpy as jnp
from jax import lax
from jax.experimental import pallas as pl
from jax.experimental.pallas import tpu as pltpu
from jax.experimental.pallas import tpu_sc as plsc

NC = 2
NS = 16
NW = NC * NS
D = 128
BW = 128
NBUF = 5


def _build(b, s):
    assert b == NW * BW and s % NBUF == 0

    mesh = plsc.VectorSubcoreMesh(core_axis_name="c", subcore_axis_name="s")

    @functools.partial(
        pl.kernel,
        out_type=jax.ShapeDtypeStruct((s, b, D), jnp.float32),
        mesh=mesh,
        compiler_params=pltpu.CompilerParams(skip_device_barrier=True),
        scratch_types=[
            pltpu.VMEM((s, BW), jnp.int32),
            pltpu.VMEM((NBUF, BW, D), jnp.float32),
        ] + [pltpu.SemaphoreType.DMA] * NBUF,
    )
    def gather_kernel(idx_hbm, table_hbm, out_hbm, idx_v, rows_v, *sems):
        wid = lax.axis_index("s") * NC + lax.axis_index("c")
        col0 = wid * BW
        pltpu.sync_copy(idx_hbm.at[:, pl.ds(col0, BW)], idx_v)

        def gather(g, buf):
            return pltpu.make_async_copy(
                table_hbm.at[idx_v.at[g]], rows_v.at[buf], sems[buf]
            )

        for buf in range(NBUF):
            gather(buf, buf).start()

        def group(i, carry):
            g0 = i * NBUF
            for buf in range(NBUF):
                g = g0 + buf
                gather(g, buf).wait()
                pltpu.sync_copy(rows_v.at[buf], out_hbm.at[g, pl.ds(col0, BW)])
                nxt = g + NBUF

                @pl.when(nxt < s)
                def _():
                    gather(nxt, buf).start()
            return carry

        lax.fori_loop(0, s // NBUF, group, None)

    return gather_kernel


@jax.jit
def kernel(x, table):
    b, s = x.shape
    p = _build(b, s)(x.T, table)
    return jnp.transpose(p, (1, 0, 2))


# speedup vs baseline: 1.0062x; 1.0006x over previous
import functools

import jax
import jax.numpy as jnp
from jax import lax
from jax.experimental import pallas as pl
from jax.experimental.pallas import tpu as pltpu
from jax.experimental.pallas import tpu_sc as plsc

NC = 2
NS = 16
NW = NC * NS
D = 128
BW = 128
NBUF = 5


def _build(b, s):
    assert b == NW * BW and s % NBUF == 0

    mesh = plsc.VectorSubcoreMesh(core_axis_name="c", subcore_axis_name="s")

    @functools.partial(
        pl.kernel,
        out_type=jax.ShapeDtypeStruct((s, b, D), jnp.float32),
        mesh=mesh,
        scratch_types=[
            pltpu.VMEM((s, BW), jnp.int32),
            pltpu.VMEM((NBUF, BW, D), jnp.float32),
        ] + [pltpu.SemaphoreType.DMA] * NBUF,
    )
    def gather_kernel(idx_hbm, table_hbm, out_hbm, idx_v, rows_v, *sems):
        wid = lax.axis_index("s") * NC + lax.axis_index("c")
        col0 = wid * BW
        pltpu.sync_copy(idx_hbm.at[:, pl.ds(col0, BW)], idx_v)

        def gather(g, buf):
            return pltpu.make_async_copy(
                table_hbm.at[idx_v.at[g]], rows_v.at[buf], sems[buf]
            )

        for buf in range(NBUF):
            gather(buf, buf).start()

        def group(i, carry):
            g0 = i * NBUF
            for buf in range(NBUF):
                g = g0 + buf
                gather(g, buf).wait()
                pltpu.sync_copy(rows_v.at[buf], out_hbm.at[g, pl.ds(col0, BW)])
                nxt = g + NBUF

                @pl.when(nxt < s)
                def _():
                    gather(nxt, buf).start()
            return carry

        lax.fori_loop(0, s // NBUF, group, None)

    return gather_kernel


@jax.jit
def kernel(x, table):
    b, s = x.shape
    p = _build(b, s)(x.T, table)
    return jnp.transpose(p, (1, 0, 2))
